# baseline (speedup 1.0000x reference)
import functools

import jax
import jax.numpy as jnp
from jax import lax
from jax.experimental import pallas as pl
from jax.experimental.pallas import tpu as pltpu
from jax.experimental.pallas import tpu_sc as plsc

N, J, D = 10000, 32, 128
L = 16
NW = 32

SC_N = 3584
TC_N = N - SC_N
CPW = SC_N // NW
NB = 8
NBLK = CPW // NB
TCB = 128
INV = 1.0 / J

_mesh = plsc.VectorSubcoreMesh(core_axis_name="c", subcore_axis_name="s")


@functools.partial(
    pl.kernel,
    mesh=_mesh,
    out_type=jax.ShapeDtypeStruct((SC_N, D), jnp.float32),
    scratch_types=[
        pltpu.VMEM((2, NB, J, D), jnp.float32),
        pltpu.VMEM((CPW, D), jnp.float32),
        pltpu.SemaphoreType.DMA,
        pltpu.SemaphoreType.DMA,
    ],
)
def _mean_sc(x_hbm, out_hbm, buf, stage, sem0, sem1):
    wid = lax.axis_index("s") * 2 + lax.axis_index("c")
    base = TC_N + wid * CPW
    sems = (sem0, sem1)

    def start(blk, slot):
        pltpu.async_copy(
            x_hbm.at[pl.ds(base + blk * NB, NB)], buf.at[slot], sems[slot])

    def wait(slot):
        pltpu.make_async_copy(
            x_hbm.at[pl.ds(0, NB)], buf.at[slot], sems[slot]).wait()

    def compute(blk, slot):
        def node_body(n, carry):
            row = blk * NB + n
            accs = [buf[slot, n, 0, pl.ds(v * L, L)] for v in range(D // L)]
            for j in range(1, J):
                for v in range(D // L):
                    accs[v] = accs[v] + buf[slot, n, j, pl.ds(v * L, L)]
            for v in range(D // L):
                stage[row, pl.ds(v * L, L)] = accs[v] * INV
            return carry

        lax.fori_loop(0, NB, node_body, 0, unroll=2)

    start(0, 0)

    def blk_pair(k, carry):
        blk0 = 2 * k
        start(blk0 + 1, 1)
        wait(0)
        compute(blk0, 0)
        blk1 = blk0 + 1
        start(jnp.minimum(blk1 + 1, NBLK - 1), 0)
        wait(1)
        compute(blk1, 1)
        return carry

    lax.fori_loop(0, NBLK // 2, blk_pair, 0)
    wait(0)
    pltpu.sync_copy(stage, out_hbm.at[pl.ds(wid * CPW, CPW)])


def _tc_body(x_ref, o_ref):
    acc = x_ref[:, 0, :]
    for j in range(1, J):
        acc = acc + x_ref[:, j, :]
    o_ref[...] = acc * INV


_tc_mean = pl.pallas_call(
    _tc_body,
    grid=(pl.cdiv(TC_N, TCB),),
    in_specs=[pl.BlockSpec((TCB, J, D), lambda i: (i, 0, 0))],
    out_specs=pl.BlockSpec((TCB, D), lambda i: (i, 0)),
    out_shape=jax.ShapeDtypeStruct((TC_N, D), jnp.float32),
)


def kernel(neighbours_features):
    sc_out = _mean_sc(neighbours_features)
    tc_out = _tc_mean(neighbours_features)
    return jnp.concatenate([tc_out, sc_out], axis=0)


# speedup vs baseline: 1.1968x; 1.1968x over previous
import functools

import jax
import jax.numpy as jnp
from jax import lax
from jax.experimental import pallas as pl
from jax.experimental.pallas import tpu as pltpu
from jax.experimental.pallas import tpu_sc as plsc

N, J, D = 10000, 32, 128
L = 16
NW = 32

SC_N = 3584
TC_N = N - SC_N
CPW = SC_N // NW
NB = 8
NBLK = CPW // NB
TCB = 256
INV = 1.0 / J

_mesh = plsc.VectorSubcoreMesh(core_axis_name="c", subcore_axis_name="s")


@functools.partial(
    pl.kernel,
    mesh=_mesh,
    out_type=jax.ShapeDtypeStruct((SC_N, D), jnp.float32),
    scratch_types=[
        pltpu.VMEM((2, NB, J, D), jnp.float32),
        pltpu.VMEM((CPW, D), jnp.float32),
        pltpu.SemaphoreType.DMA,
        pltpu.SemaphoreType.DMA,
    ],
)
def _mean_sc(x_hbm, out_hbm, buf, stage, sem0, sem1):
    wid = lax.axis_index("s") * 2 + lax.axis_index("c")
    base = TC_N + wid * CPW
    sems = (sem0, sem1)

    def start(blk, slot):
        pltpu.async_copy(
            x_hbm.at[pl.ds(base + blk * NB, NB)], buf.at[slot], sems[slot])

    def wait(slot):
        pltpu.make_async_copy(
            x_hbm.at[pl.ds(0, NB)], buf.at[slot], sems[slot]).wait()

    def compute(blk, slot):
        def node_body(n, carry):
            row = blk * NB + n
            accs = [buf[slot, n, 0, pl.ds(v * L, L)] for v in range(D // L)]
            for j in range(1, J):
                for v in range(D // L):
                    accs[v] = accs[v] + buf[slot, n, j, pl.ds(v * L, L)]
            for v in range(D // L):
                stage[row, pl.ds(v * L, L)] = accs[v] * INV
            return carry

        lax.fori_loop(0, NB, node_body, 0, unroll=2)

    start(0, 0)

    def blk_pair(k, carry):
        blk0 = 2 * k
        start(blk0 + 1, 1)
        wait(0)
        compute(blk0, 0)
        blk1 = blk0 + 1
        start(jnp.minimum(blk1 + 1, NBLK - 1), 0)
        wait(1)
        compute(blk1, 1)
        return carry

    lax.fori_loop(0, NBLK // 2, blk_pair, 0)
    wait(0)
    pltpu.sync_copy(stage, out_hbm.at[pl.ds(wid * CPW, CPW)])


def _tc_body(x_ref, o_ref):
    o_ref[...] = jnp.sum(x_ref[...], axis=1) * INV


_tc_mean = pl.pallas_call(
    _tc_body,
    grid=(pl.cdiv(TC_N, TCB),),
    in_specs=[pl.BlockSpec((TCB, J, D), lambda i: (i, 0, 0))],
    out_specs=pl.BlockSpec((TCB, D), lambda i: (i, 0)),
    out_shape=jax.ShapeDtypeStruct((TC_N, D), jnp.float32),
)


def kernel(neighbours_features):
    sc_out = _mean_sc(neighbours_features)
    tc_out = _tc_mean(neighbours_features)
    return jnp.concatenate([tc_out, sc_out], axis=0)


# speedup vs baseline: 1.7826x; 1.4894x over previous
import jax
import jax.numpy as jnp
from jax.experimental import pallas as pl

N, J, D = 10000, 32, 128
TCB = 512
INV = 1.0 / J


def _mean_body(x_ref, o_ref):
    o_ref[...] = jnp.sum(x_ref[...], axis=1) * INV


_mean = pl.pallas_call(
    _mean_body,
    grid=(pl.cdiv(N, TCB),),
    in_specs=[pl.BlockSpec((TCB, J, D), lambda i: (i, 0, 0))],
    out_specs=pl.BlockSpec((TCB, D), lambda i: (i, 0)),
    out_shape=jax.ShapeDtypeStruct((N, D), jnp.float32),
)


def kernel(neighbours_features):
    return _mean(neighbours_features)
